# baseline (speedup 1.0000x reference)
import functools

import jax
import jax.numpy as jnp
from jax import lax
from jax.experimental import pallas as pl
from jax.experimental.pallas import tpu as pltpu
from jax.experimental.pallas import tpu_sc as plsc

BATCH = 16384
EMB_DIM = 64
SUB = 8
NUM_WORKERS = 32
ROWS_PER_WORKER = BATCH // NUM_WORKERS
LANES = 16
NUM_GROUPS = ROWS_PER_WORKER // LANES
CHUNK = 64
NUM_CHUNKS = ROWS_PER_WORKER // CHUNK
GROUPS_PER_CHUNK = CHUNK // LANES
CHUNK_BYTES = CHUNK * SUB * EMB_DIM * 4


def _sc_body(uid_h, sid_h, pid_h, u_emb_h, v_emb_h, w_h, b_h,
             score_h, pol_h,
             uid_v, sid_v, pid_v, slab_v, v_rows, drain_v,
             score_v, pol_v, w_v, b_v,
             sem_idx, sem_u, sem_v, sem_w):
    wid = lax.axis_index("s") * 2 + lax.axis_index("c")
    base = wid * ROWS_PER_WORKER

    pltpu.async_copy(uid_h.at[pl.ds(base, ROWS_PER_WORKER)], uid_v, sem_idx)
    pltpu.async_copy(sid_h.at[pl.ds(base, ROWS_PER_WORKER)], sid_v, sem_idx)
    h_idx = pltpu.async_copy(
        pid_h.at[pl.ds(base, ROWS_PER_WORKER)], pid_v, sem_idx)
    h_w1 = pltpu.async_copy(w_h, w_v, sem_w)
    h_w2 = pltpu.async_copy(b_h, b_v, sem_w)
    h_idx.wait()
    h_idx.wait()
    h_idx.wait()

    def issue_v(g, carry):
        off = pl.multiple_of(g * LANES, LANES)
        svec = sid_v[pl.ds(off, LANES)] * EMB_DIM
        for j in range(LANES):
            start = pl.multiple_of(svec[j], EMB_DIM)
            dst = pl.multiple_of((off + j) * EMB_DIM, EMB_DIM)
            pltpu.async_copy(v_emb_h.at[pl.ds(start, EMB_DIM)],
                             v_rows.at[pl.ds(dst, EMB_DIM)], sem_v)
        return carry

    lax.fori_loop(0, NUM_GROUPS, issue_v, 0)
    pltpu.make_async_copy(
        v_emb_h.at[pl.ds(0, ROWS_PER_WORKER * EMB_DIM)], drain_v, sem_v).wait()
    h_w1.wait()
    h_w2.wait()

    bias = b_v[...]
    zeros = jnp.zeros((LANES,), jnp.float32)
    lane_iota = lax.iota(jnp.int32, LANES)
    seven = jnp.full((LANES,), 7, jnp.int32)

    def phase_body(p, carry):
        is_score = p == 0
        mask16 = jnp.broadcast_to(is_score, (LANES,))

        def chunk_body(q, c):
            coff = pl.multiple_of(q * CHUNK, CHUNK)

            def issue_u(g, c2):
                off = pl.multiple_of(g * LANES, LANES)
                sl = pl.ds(coff + off, LANES)
                rvec = lax.select(mask16, uid_v[sl], pid_v[sl])
                bvec = jnp.bitwise_and(rvec, jnp.int32(~7))
                for j in range(LANES):
                    st = pl.multiple_of(bvec[j], SUB)
                    pltpu.async_copy(u_emb_h.at[pl.ds(st, SUB)],
                                     slab_v.at[off + j], sem_u)
                return c2

            lax.fori_loop(0, GROUPS_PER_CHUNK, issue_u, 0)
            pltpu.make_async_copy(
                v_emb_h.at[pl.ds(0, CHUNK * SUB * EMB_DIM)],
                drain_v.at[pl.ds(0, CHUNK * SUB * EMB_DIM)], sem_u).wait()
            pltpu.make_async_copy(
                v_emb_h.at[pl.ds(0, CHUNK * SUB * EMB_DIM)],
                drain_v.at[pl.ds(0, CHUNK * SUB * EMB_DIM)], sem_u).wait()

            def group_body(g, c2):
                off = pl.multiple_of(g * LANES, LANES)
                sl = pl.ds(coff + off, LANES)
                rvec = lax.select(mask16, uid_v[sl], pid_v[sl])
                svec = jnp.bitwise_and(rvec, seven)
                lvec = off + lane_iota
                vbase = (coff + off + lane_iota) * EMB_DIM

                def col_body(col, acc):
                    cvec = jnp.full((LANES,), 0, jnp.int32) + col
                    uu = plsc.load_gather(slab_v, [lvec, svec, cvec])
                    vv = plsc.load_gather(v_rows, [vbase + cvec])
                    wc = plsc.load_gather(w_v, [cvec])
                    other = lax.select(mask16, vv, wc)
                    return acc + uu * other

                acc = lax.fori_loop(0, EMB_DIM, col_body, zeros)
                out_slice = pl.ds(pl.multiple_of(coff + off, LANES), LANES)

                @pl.when(is_score)
                def _():
                    score_v[out_slice] = 1.0 / (1.0 + jnp.exp(-acc))

                @pl.when(jnp.logical_not(is_score))
                def _():
                    pol_v[out_slice] = 1.0 / (1.0 + jnp.exp(-(acc + bias)))

                return c2

            lax.fori_loop(0, GROUPS_PER_CHUNK, group_body, 0)
            return c

        lax.fori_loop(0, NUM_CHUNKS, chunk_body, 0)
        return carry

    lax.fori_loop(0, 2, phase_body, 0)

    pltpu.sync_copy(score_v, score_h.at[pl.ds(base, ROWS_PER_WORKER)])
    pltpu.sync_copy(pol_v, pol_h.at[pl.ds(base, ROWS_PER_WORKER)])


@jax.jit
def _run(user_id, subreddit_id, political_user_ids, u_emb, v_flat, w, b16):
    mesh = plsc.VectorSubcoreMesh(core_axis_name="c", subcore_axis_name="s")
    f32 = jnp.float32
    call = functools.partial(
        pl.kernel,
        mesh=mesh,
        out_type=[
            jax.ShapeDtypeStruct((BATCH,), f32),
            jax.ShapeDtypeStruct((BATCH,), f32),
        ],
        scratch_types=[
            pltpu.VMEM((ROWS_PER_WORKER,), jnp.int32),
            pltpu.VMEM((ROWS_PER_WORKER,), jnp.int32),
            pltpu.VMEM((ROWS_PER_WORKER,), jnp.int32),
            pltpu.VMEM((CHUNK, SUB, EMB_DIM), f32),
            pltpu.VMEM((ROWS_PER_WORKER * EMB_DIM,), f32),
            pltpu.VMEM((ROWS_PER_WORKER * EMB_DIM,), f32),
            pltpu.VMEM((ROWS_PER_WORKER,), f32),
            pltpu.VMEM((ROWS_PER_WORKER,), f32),
            pltpu.VMEM((EMB_DIM,), f32),
            pltpu.VMEM((LANES,), f32),
            pltpu.SemaphoreType.DMA,
            pltpu.SemaphoreType.DMA,
            pltpu.SemaphoreType.DMA,
            pltpu.SemaphoreType.DMA,
        ],
        compiler_params=pltpu.CompilerParams(needs_layout_passes=False),
    )
    return call(_sc_body)(user_id, subreddit_id, political_user_ids,
                          u_emb, v_flat, w, b16)


def kernel(user_id, subreddit_id, political_user_ids, u_emb, v_emb, pol_W, pol_b):
    w = pol_W.reshape(EMB_DIM)
    b16 = jnp.broadcast_to(pol_b, (LANES,))
    v_flat = v_emb.reshape(-1)
    score, pol = _run(user_id.astype(jnp.int32), subreddit_id.astype(jnp.int32),
                      political_user_ids.astype(jnp.int32), u_emb, v_flat, w, b16)
    return score, pol.reshape(BATCH, 1)


# speedup vs baseline: 1.1631x; 1.1631x over previous
import functools

import jax
import jax.numpy as jnp
from jax import lax
from jax.experimental import pallas as pl
from jax.experimental.pallas import tpu as pltpu
from jax.experimental.pallas import tpu_sc as plsc

BATCH = 16384
EMB_DIM = 64
NUM_WORKERS = 32
ROWS_PER_WORKER = BATCH // NUM_WORKERS
LANES = 16
NUM_GROUPS = ROWS_PER_WORKER // LANES


def _sc_body(uid_h, sid_h, pid_h, u_emb_h, v_flat_h, w_h, b_h,
             score_h, pol_h,
             uid_v, sid_v, pid_v, u_rows, v_rows, drain_v,
             score_v, pol_v, w_v, b_v,
             sem_idx, sem_u, sem_v, sem_w):
    wid = lax.axis_index("s") * 2 + lax.axis_index("c")
    base = wid * ROWS_PER_WORKER

    pltpu.async_copy(uid_h.at[pl.ds(base, ROWS_PER_WORKER)], uid_v, sem_idx)
    pltpu.async_copy(sid_h.at[pl.ds(base, ROWS_PER_WORKER)], sid_v, sem_idx)
    h_idx = pltpu.async_copy(
        pid_h.at[pl.ds(base, ROWS_PER_WORKER)], pid_v, sem_idx)
    h_w1 = pltpu.async_copy(w_h, w_v, sem_w)
    h_w2 = pltpu.async_copy(b_h, b_v, sem_w)
    h_idx.wait()
    h_idx.wait()
    h_idx.wait()

    def issue_v(g, carry):
        off = pl.multiple_of(g * LANES, LANES)
        svec = sid_v[pl.ds(off, LANES)] * EMB_DIM
        for j in range(LANES):
            start = pl.multiple_of(svec[j], EMB_DIM)
            dst = pl.multiple_of((off + j) * EMB_DIM, EMB_DIM)
            pltpu.async_copy(v_flat_h.at[pl.ds(start, EMB_DIM)],
                             v_rows.at[pl.ds(dst, EMB_DIM)], sem_v)
        return carry

    lax.fori_loop(0, NUM_GROUPS, issue_v, 0)
    pltpu.make_async_copy(
        v_flat_h.at[pl.ds(0, ROWS_PER_WORKER * EMB_DIM)], drain_v,
        sem_v).wait()
    h_w1.wait()
    h_w2.wait()

    bias = b_v[...]
    zeros = jnp.zeros((LANES,), jnp.float32)
    lane_iota = lax.iota(jnp.int32, LANES)

    def phase_body(p, carry):
        is_score = p == 0
        mask16 = jnp.broadcast_to(is_score, (LANES,))

        def issue_u(g, c):
            off = pl.multiple_of(g * LANES, LANES)
            sl = pl.ds(off, LANES)
            rvec = lax.select(mask16, uid_v[sl], pid_v[sl])
            for j in range(LANES):
                pltpu.async_copy(u_emb_h.at[rvec[j]], u_rows.at[off + j],
                                 sem_u)
            return c

        lax.fori_loop(0, NUM_GROUPS, issue_u, 0)
        pltpu.make_async_copy(
            v_flat_h.at[pl.ds(0, ROWS_PER_WORKER * EMB_DIM)], drain_v,
            sem_u).wait()
        pltpu.make_async_copy(
            v_flat_h.at[pl.ds(0, ROWS_PER_WORKER * EMB_DIM)], drain_v,
            sem_u).wait()

        def group_body(g, c):
            off = pl.multiple_of(g * LANES, LANES)
            rows = off + lane_iota
            vbase = rows * EMB_DIM
            out_slice = pl.ds(pl.multiple_of(off, LANES), LANES)

            @pl.when(is_score)
            def _():
                def col_body(ci, accs):
                    a0, a1 = accs
                    cb = ci * 4
                    for k in range(4):
                        cvec = jnp.full((LANES,), 0, jnp.int32) + (cb + k)
                        uu = plsc.load_gather(u_rows, [rows, cvec])
                        vv = plsc.load_gather(v_rows, [vbase + cvec])
                        if k % 2 == 0:
                            a0 = a0 + uu * vv
                        else:
                            a1 = a1 + uu * vv
                    return a0, a1

                a0, a1 = lax.fori_loop(0, EMB_DIM // 4, col_body,
                                       (zeros, zeros))
                score_v[out_slice] = 1.0 / (1.0 + jnp.exp(-(a0 + a1)))

            @pl.when(jnp.logical_not(is_score))
            def _():
                def col_body(ci, accs):
                    a0, a1 = accs
                    cb = ci * 4
                    for k in range(4):
                        cvec = jnp.full((LANES,), 0, jnp.int32) + (cb + k)
                        uu = plsc.load_gather(u_rows, [rows, cvec])
                        wc = plsc.load_gather(w_v, [cvec])
                        if k % 2 == 0:
                            a0 = a0 + uu * wc
                        else:
                            a1 = a1 + uu * wc
                    return a0, a1

                a0, a1 = lax.fori_loop(0, EMB_DIM // 4, col_body,
                                       (zeros, zeros))
                pol_v[out_slice] = 1.0 / (1.0 + jnp.exp(-(a0 + a1 + bias)))

            return c

        lax.fori_loop(0, NUM_GROUPS, group_body, 0)
        return carry

    lax.fori_loop(0, 2, phase_body, 0)

    pltpu.sync_copy(score_v, score_h.at[pl.ds(base, ROWS_PER_WORKER)])
    pltpu.sync_copy(pol_v, pol_h.at[pl.ds(base, ROWS_PER_WORKER)])


@jax.jit
def _run(user_id, subreddit_id, political_user_ids, u_emb, v_flat, w, b16):
    mesh = plsc.VectorSubcoreMesh(core_axis_name="c", subcore_axis_name="s")
    f32 = jnp.float32
    call = functools.partial(
        pl.kernel,
        mesh=mesh,
        out_type=[
            jax.ShapeDtypeStruct((BATCH,), f32),
            jax.ShapeDtypeStruct((BATCH,), f32),
        ],
        scratch_types=[
            pltpu.VMEM((ROWS_PER_WORKER,), jnp.int32),
            pltpu.VMEM((ROWS_PER_WORKER,), jnp.int32),
            pltpu.VMEM((ROWS_PER_WORKER,), jnp.int32),
            pltpu.VMEM((ROWS_PER_WORKER, EMB_DIM), f32),
            pltpu.VMEM((ROWS_PER_WORKER * EMB_DIM,), f32),
            pltpu.VMEM((ROWS_PER_WORKER * EMB_DIM,), f32),
            pltpu.VMEM((ROWS_PER_WORKER,), f32),
            pltpu.VMEM((ROWS_PER_WORKER,), f32),
            pltpu.VMEM((EMB_DIM,), f32),
            pltpu.VMEM((LANES,), f32),
            pltpu.SemaphoreType.DMA,
            pltpu.SemaphoreType.DMA,
            pltpu.SemaphoreType.DMA,
            pltpu.SemaphoreType.DMA,
        ],
        compiler_params=pltpu.CompilerParams(needs_layout_passes=False),
    )
    return call(_sc_body)(user_id, subreddit_id, political_user_ids,
                          u_emb, v_flat, w, b16)


def kernel(user_id, subreddit_id, political_user_ids, u_emb, v_emb, pol_W, pol_b):
    w = pol_W.reshape(EMB_DIM)
    b16 = jnp.broadcast_to(pol_b, (LANES,))
    v_flat = v_emb.reshape(-1)
    score, pol = _run(user_id.astype(jnp.int32), subreddit_id.astype(jnp.int32),
                      political_user_ids.astype(jnp.int32), u_emb, v_flat, w, b16)
    return score, pol.reshape(BATCH, 1)


# speedup vs baseline: 1.1668x; 1.0032x over previous
import functools

import jax
import jax.numpy as jnp
from jax import lax
from jax.experimental import pallas as pl
from jax.experimental.pallas import tpu as pltpu
from jax.experimental.pallas import tpu_sc as plsc

BATCH = 16384
EMB_DIM = 64
NUM_WORKERS = 32
ROWS_PER_WORKER = BATCH // NUM_WORKERS
LANES = 16
NUM_GROUPS = ROWS_PER_WORKER // LANES


def _sc_body(uid_h, sid_h, pid_h, u_emb_h, v_flat_h, w_h, b_h,
             score_h, pol_h,
             uid_v, sid_v, pid_v, u_rows, v_rows,
             score_v, pol_v, w_v, b_v,
             sem_idx, sem_u, sem_v, sem_w):
    wid = lax.axis_index("s") * 2 + lax.axis_index("c")
    base = wid * ROWS_PER_WORKER

    pltpu.async_copy(uid_h.at[pl.ds(base, ROWS_PER_WORKER)], uid_v, sem_idx)
    pltpu.async_copy(sid_h.at[pl.ds(base, ROWS_PER_WORKER)], sid_v, sem_idx)
    h_idx = pltpu.async_copy(
        pid_h.at[pl.ds(base, ROWS_PER_WORKER)], pid_v, sem_idx)
    h_w1 = pltpu.async_copy(w_h, w_v, sem_w)
    h_w2 = pltpu.async_copy(b_h, b_v, sem_w)
    h_idx.wait()
    h_idx.wait()
    h_idx.wait()

    def issue_v(g, carry):
        off = pl.multiple_of(g * LANES, LANES)
        svec = sid_v[pl.ds(off, LANES)] * EMB_DIM
        for j in range(LANES):
            start = pl.multiple_of(svec[j], EMB_DIM)
            dst = pl.multiple_of((off + j) * EMB_DIM, EMB_DIM)
            pltpu.async_copy(v_flat_h.at[pl.ds(start, EMB_DIM)],
                             v_rows.at[pl.ds(dst, EMB_DIM)], sem_v)
        return carry

    lax.fori_loop(0, NUM_GROUPS, issue_v, 0)
    pltpu.make_async_copy(
        v_flat_h.at[pl.ds(0, ROWS_PER_WORKER * EMB_DIM)], v_rows,
        sem_v).wait()
    h_w1.wait()
    h_w2.wait()

    bias = b_v[...]
    zeros = jnp.zeros((LANES,), jnp.float32)
    lane_iota = lax.iota(jnp.int32, LANES)

    HALF = ROWS_PER_WORKER // 2
    HGROUPS = HALF // LANES

    def step_body(p, carry):
        @pl.when(p >= 1)
        def _():
            pltpu.make_async_copy(
                v_flat_h.at[pl.ds(0, HALF * EMB_DIM)],
                v_rows.at[pl.ds(0, HALF * EMB_DIM)], sem_u).wait()
            pltpu.make_async_copy(
                v_flat_h.at[pl.ds(0, HALF * EMB_DIM)],
                v_rows.at[pl.ds(0, HALF * EMB_DIM)], sem_u).wait()

        @pl.when(p < 4)
        def _():
            f_is_score = p < 2
            fmask16 = jnp.broadcast_to(f_is_score, (LANES,))
            fhalf = jnp.bitwise_and(p, 1) * HALF
            fbuf = jnp.bitwise_and(p, 1)

            def issue_u(g, c):
                off = pl.multiple_of(g * LANES, LANES)
                sl = pl.ds(fhalf + off, LANES)
                rvec = lax.select(fmask16, uid_v[sl], pid_v[sl])
                for j in range(LANES):
                    pltpu.async_copy(u_emb_h.at[rvec[j]],
                                     u_rows.at[fbuf, off + j], sem_u)
                return c

            lax.fori_loop(0, HGROUPS, issue_u, 0)

        @pl.when(p >= 1)
        def _():
            q = p - 1
            is_score = q < 2
            chalf = jnp.bitwise_and(q, 1) * HALF
            cbuf16 = jnp.full((LANES,), 0, jnp.int32) + jnp.bitwise_and(q, 1)

            def group_body(g, c):
                off = pl.multiple_of(g * LANES, LANES)
                rows = off + lane_iota
                grows = chalf + rows
                vbase = grows * EMB_DIM
                out_slice = pl.ds(chalf + off, LANES)

                @pl.when(is_score)
                def _():
                    def col_body(ci, accs):
                        a0, a1 = accs
                        cb = ci * 4
                        for k in range(4):
                            cvec = jnp.full((LANES,), 0, jnp.int32) + (cb + k)
                            uu = plsc.load_gather(
                                u_rows, [cbuf16, rows, cvec])
                            vv = plsc.load_gather(v_rows, [vbase + cvec])
                            if k % 2 == 0:
                                a0 = a0 + uu * vv
                            else:
                                a1 = a1 + uu * vv
                        return a0, a1

                    a0, a1 = lax.fori_loop(0, EMB_DIM // 4, col_body,
                                           (zeros, zeros))
                    score_v[out_slice] = 1.0 / (1.0 + jnp.exp(-(a0 + a1)))

                @pl.when(jnp.logical_not(is_score))
                def _():
                    def col_body(ci, accs):
                        a0, a1 = accs
                        cb = ci * 4
                        for k in range(4):
                            cvec = jnp.full((LANES,), 0, jnp.int32) + (cb + k)
                            uu = plsc.load_gather(
                                u_rows, [cbuf16, rows, cvec])
                            wc = plsc.load_gather(w_v, [cvec])
                            if k % 2 == 0:
                                a0 = a0 + uu * wc
                            else:
                                a1 = a1 + uu * wc
                        return a0, a1

                    a0, a1 = lax.fori_loop(0, EMB_DIM // 4, col_body,
                                           (zeros, zeros))
                    pol_v[out_slice] = 1.0 / (1.0 + jnp.exp(-(a0 + a1 + bias)))

                return c

            lax.fori_loop(0, HGROUPS, group_body, 0)

        return carry

    lax.fori_loop(0, 5, step_body, 0)

    pltpu.sync_copy(score_v, score_h.at[pl.ds(base, ROWS_PER_WORKER)])
    pltpu.sync_copy(pol_v, pol_h.at[pl.ds(base, ROWS_PER_WORKER)])


@jax.jit
def _run(user_id, subreddit_id, political_user_ids, u_emb, v_flat, w, b16):
    mesh = plsc.VectorSubcoreMesh(core_axis_name="c", subcore_axis_name="s")
    f32 = jnp.float32
    call = functools.partial(
        pl.kernel,
        mesh=mesh,
        out_type=[
            jax.ShapeDtypeStruct((BATCH,), f32),
            jax.ShapeDtypeStruct((BATCH,), f32),
        ],
        scratch_types=[
            pltpu.VMEM((ROWS_PER_WORKER,), jnp.int32),
            pltpu.VMEM((ROWS_PER_WORKER,), jnp.int32),
            pltpu.VMEM((ROWS_PER_WORKER,), jnp.int32),
            pltpu.VMEM((2, ROWS_PER_WORKER // 2, EMB_DIM), f32),
            pltpu.VMEM((ROWS_PER_WORKER * EMB_DIM,), f32),
            pltpu.VMEM((ROWS_PER_WORKER,), f32),
            pltpu.VMEM((ROWS_PER_WORKER,), f32),
            pltpu.VMEM((EMB_DIM,), f32),
            pltpu.VMEM((LANES,), f32),
            pltpu.SemaphoreType.DMA,
            pltpu.SemaphoreType.DMA,
            pltpu.SemaphoreType.DMA,
            pltpu.SemaphoreType.DMA,
        ],
        compiler_params=pltpu.CompilerParams(needs_layout_passes=False),
    )
    return call(_sc_body)(user_id, subreddit_id, political_user_ids,
                          u_emb, v_flat, w, b16)


def kernel(user_id, subreddit_id, political_user_ids, u_emb, v_emb, pol_W, pol_b):
    w = pol_W.reshape(EMB_DIM)
    b16 = jnp.broadcast_to(pol_b, (LANES,))
    v_flat = v_emb.reshape(-1)
    score, pol = _run(user_id.astype(jnp.int32), subreddit_id.astype(jnp.int32),
                      political_user_ids.astype(jnp.int32), u_emb, v_flat, w, b16)
    return score, pol.reshape(BATCH, 1)
